# Initial kernel scaffold; baseline (speedup 1.0000x reference)
import jax
import jax.numpy as jnp
from jax.experimental import pallas as pl


def kernel(id, node_embedding, weight):
    raise NotImplementedError("write your pallas kernel here")


# speedup vs baseline: 1.1884x; 1.1884x over previous
import functools

import jax
import jax.numpy as jnp
from jax import lax
from jax.experimental import pallas as pl
from jax.experimental.pallas import tpu as pltpu
from jax.experimental.pallas import tpu_sc as plsc

SIZE = 100000
D = 128
B = 50000
NW = 32
CHUNK = 400
NCHUNKS = B // CHUNK
MAX_CHUNKS_PER_WORKER = -(-NCHUNKS // NW)
LANES = 16


def _body(idx_hbm, node_hbm, weight_hbm, out_hbm, idx_v, node_v, rows_v, sem):
    wid = lax.axis_index("s") * 2 + lax.axis_index("c")
    for k in range(MAX_CHUNKS_PER_WORKER):
        c = wid + k * NW

        @pl.when(c < NCHUNKS)
        def _():
            base = c * CHUNK
            pltpu.sync_copy(idx_hbm.at[pl.ds(base, CHUNK)], idx_v)
            pltpu.sync_copy(node_hbm.at[pl.ds(base, CHUNK)], node_v)
            pltpu.async_copy(weight_hbm.at[idx_v], rows_v, sem).wait()

            def addrow(j, carry):
                for l in range(D // LANES):
                    sl = pl.ds(l * LANES, LANES)
                    plsc.addupdate(rows_v.at[j, sl], node_v[j, sl])
                return carry

            lax.fori_loop(0, CHUNK, addrow, 0)
            pltpu.sync_copy(rows_v, out_hbm.at[pl.ds(base, CHUNK)])


@jax.jit
def _run(id, node_embedding, weight):
    mesh = plsc.VectorSubcoreMesh(core_axis_name="c", subcore_axis_name="s")
    f = pl.kernel(
        _body,
        out_type=jax.ShapeDtypeStruct((B, D), jnp.float32),
        mesh=mesh,
        scratch_types=[
            pltpu.VMEM((CHUNK,), jnp.int32),
            pltpu.VMEM((CHUNK, D), jnp.float32),
            pltpu.VMEM((CHUNK, D), jnp.float32),
            pltpu.SemaphoreType.DMA,
        ],
    )
    return f(id, node_embedding, weight)


def kernel(id, node_embedding, weight):
    return _run(id.astype(jnp.int32), node_embedding, weight)


# speedup vs baseline: 1.5155x; 1.2752x over previous
import functools

import jax
import jax.numpy as jnp
from jax import lax
from jax.experimental import pallas as pl
from jax.experimental.pallas import tpu as pltpu
from jax.experimental.pallas import tpu_sc as plsc

SIZE = 100000
D = 128
B = 50000
NW = 32
CHUNK = 400
NCHUNKS = B // CHUNK
MAX_CHUNKS_PER_WORKER = -(-NCHUNKS // NW)
LANES = 16


def _body(idx_hbm, node_hbm, weight_hbm, out_hbm, idx_v, node_v, rows_v, sem):
    wid = lax.axis_index("s") * 2 + lax.axis_index("c")
    for k in range(MAX_CHUNKS_PER_WORKER):
        c = wid + k * NW

        @pl.when(c < NCHUNKS)
        def _():
            base = c * CHUNK
            pltpu.sync_copy(idx_hbm.at[pl.ds(base, CHUNK)], idx_v)
            pltpu.sync_copy(node_hbm.at[pl.ds(base, CHUNK)], rows_v)
            pltpu.async_copy(weight_hbm.at[idx_v], rows_v, sem, add=True).wait()
            pltpu.sync_copy(rows_v, out_hbm.at[pl.ds(base, CHUNK)])


@jax.jit
def _run(id, node_embedding, weight):
    mesh = plsc.VectorSubcoreMesh(core_axis_name="c", subcore_axis_name="s")
    f = pl.kernel(
        _body,
        out_type=jax.ShapeDtypeStruct((B, D), jnp.float32),
        mesh=mesh,
        scratch_types=[
            pltpu.VMEM((CHUNK,), jnp.int32),
            pltpu.VMEM((CHUNK, D), jnp.float32),
            pltpu.VMEM((CHUNK, D), jnp.float32),
            pltpu.SemaphoreType.DMA,
        ],
    )
    return f(id, node_embedding, weight)


def kernel(id, node_embedding, weight):
    return _run(id.astype(jnp.int32), node_embedding, weight)


# speedup vs baseline: 1.6090x; 1.0617x over previous
import jax
import jax.numpy as jnp
from jax import lax
from jax.experimental import pallas as pl
from jax.experimental.pallas import tpu as pltpu
from jax.experimental.pallas import tpu_sc as plsc

D = 128
B = 50000
NW = 32
CHUNK = 400
NCHUNKS = B // CHUNK
K = -(-NCHUNKS // NW)
NBUF = 2


def _body(idx_hbm, node_hbm, weight_hbm, out_hbm,
          idx_v0, idx_v1, rows_v0, rows_v1,
          sem_in0, sem_in1, sem_out0, sem_out1, sem_g):
    wid = lax.axis_index("s") * 2 + lax.axis_index("c")
    idx_v = (idx_v0, idx_v1)
    rows_v = (rows_v0, rows_v1)
    sem_in = (sem_in0, sem_in1)
    sem_out = (sem_out0, sem_out1)

    def valid(k):
        return (wid + k * NW) < NCHUNKS

    def in_descs(k):
        b = k % NBUF
        base = (wid + k * NW) * CHUNK
        return (
            pltpu.make_async_copy(
                idx_hbm.at[pl.ds(base, CHUNK)], idx_v[b], sem_in[b]),
            pltpu.make_async_copy(
                node_hbm.at[pl.ds(base, CHUNK)], rows_v[b], sem_in[b]),
        )

    def out_desc(k):
        b = k % NBUF
        base = (wid + k * NW) * CHUNK
        return pltpu.make_async_copy(
            rows_v[b], out_hbm.at[pl.ds(base, CHUNK)], sem_out[b])

    @pl.when(valid(0))
    def _():
        di, dn = in_descs(0)
        di.start()
        dn.start()

    for k in range(K):
        if k >= 1:
            @pl.when(valid(k - 1))
            def _(k=k):
                out_desc(k - 1).wait()

        if k + 1 < K:
            @pl.when(valid(k + 1))
            def _(k=k):
                di, dn = in_descs(k + 1)
                di.start()
                dn.start()

        @pl.when(valid(k))
        def _(k=k):
            di, dn = in_descs(k)
            di.wait()
            dn.wait()
            b = k % NBUF
            pltpu.async_copy(
                weight_hbm.at[idx_v[b]], rows_v[b], sem_g, add=True
            ).wait()
            out_desc(k).start()

    @pl.when(valid(K - 1))
    def _():
        out_desc(K - 1).wait()


@jax.jit
def _run(id, node_embedding, weight):
    mesh = plsc.VectorSubcoreMesh(core_axis_name="c", subcore_axis_name="s")
    f = pl.kernel(
        _body,
        out_type=jax.ShapeDtypeStruct((B, D), jnp.float32),
        mesh=mesh,
        scratch_types=[
            pltpu.VMEM((CHUNK,), jnp.int32),
            pltpu.VMEM((CHUNK,), jnp.int32),
            pltpu.VMEM((CHUNK, D), jnp.float32),
            pltpu.VMEM((CHUNK, D), jnp.float32),
            pltpu.SemaphoreType.DMA,
            pltpu.SemaphoreType.DMA,
            pltpu.SemaphoreType.DMA,
            pltpu.SemaphoreType.DMA,
            pltpu.SemaphoreType.DMA,
        ],
    )
    return f(id, node_embedding, weight)


def kernel(id, node_embedding, weight):
    return _run(id.astype(jnp.int32), node_embedding, weight)


# speedup vs baseline: 1.6484x; 1.0245x over previous
import jax
import jax.numpy as jnp
from jax import lax
from jax.experimental import pallas as pl
from jax.experimental.pallas import tpu as pltpu
from jax.experimental.pallas import tpu_sc as plsc

D = 128
B = 50000
NW = 32
CHUNK = 200
NCHUNKS = B // CHUNK
K = -(-NCHUNKS // NW)
NBUF = 4


def _body(idx_hbm, node_hbm, weight_hbm, out_hbm, *scratch):
    idx_v = scratch[0:NBUF]
    rows_v = scratch[NBUF:2 * NBUF]
    sem_in = scratch[2 * NBUF:3 * NBUF]
    sem_out = scratch[3 * NBUF:4 * NBUF]
    sem_g = scratch[4 * NBUF]
    wid = lax.axis_index("s") * 2 + lax.axis_index("c")

    def valid(k):
        return (wid + k * NW) < NCHUNKS

    def in_descs(k):
        b = k % NBUF
        base = (wid + k * NW) * CHUNK
        return (
            pltpu.make_async_copy(
                idx_hbm.at[pl.ds(base, CHUNK)], idx_v[b], sem_in[b]),
            pltpu.make_async_copy(
                node_hbm.at[pl.ds(base, CHUNK)], rows_v[b], sem_in[b]),
        )

    def out_desc(k):
        b = k % NBUF
        base = (wid + k * NW) * CHUNK
        return pltpu.make_async_copy(
            rows_v[b], out_hbm.at[pl.ds(base, CHUNK)], sem_out[b])

    def start_in(k):
        if k < K:
            @pl.when(valid(k))
            def _():
                di, dn = in_descs(k)
                di.start()
                dn.start()

    for k in range(NBUF - 2):
        start_in(k)

    for k in range(K):
        if k >= 2:
            @pl.when(valid(k - 2))
            def _(k=k):
                out_desc(k - 2).wait()

        start_in(k + NBUF - 2)

        @pl.when(valid(k))
        def _(k=k):
            di, dn = in_descs(k)
            di.wait()
            dn.wait()
            b = k % NBUF
            pltpu.async_copy(
                weight_hbm.at[idx_v[b]], rows_v[b], sem_g, add=True
            ).wait()
            out_desc(k).start()

    for k in range(max(K - 2, 0), K):
        @pl.when(valid(k))
        def _(k=k):
            out_desc(k).wait()


@jax.jit
def _run(id, node_embedding, weight):
    mesh = plsc.VectorSubcoreMesh(core_axis_name="c", subcore_axis_name="s")
    f = pl.kernel(
        _body,
        out_type=jax.ShapeDtypeStruct((B, D), jnp.float32),
        mesh=mesh,
        scratch_types=(
            [pltpu.VMEM((CHUNK,), jnp.int32) for _ in range(NBUF)]
            + [pltpu.VMEM((CHUNK, D), jnp.float32) for _ in range(NBUF)]
            + [pltpu.SemaphoreType.DMA for _ in range(2 * NBUF + 1)]
        ),
    )
    return f(id, node_embedding, weight)


def kernel(id, node_embedding, weight):
    return _run(id.astype(jnp.int32), node_embedding, weight)
